# initial kernel scaffold (unmeasured)
import jax
import jax.numpy as jnp
from jax import lax
from jax.experimental import pallas as pl
from jax.experimental.pallas import tpu as pltpu

N_DEV = 4
SQ = 2048
SKV = 2048
HL = 8
HT = 32
DH = 128
NR = 4
QR = SQ // NR
KR = SKV // NR
DM = 1024
SCALE = 0.08838834764831843
F32 = jnp.float32


def _qproj_body(x_ref, wq_ref, q_ref):
    qh = jnp.dot(x_ref[0], wq_ref[...], preferred_element_type=F32)
    qh = qh.reshape(8, NR, 64, DH).transpose(1, 0, 2, 3)
    q_ref[...] = qh.reshape(1, NR, QR, DH)


def _regroup_body(k_ref, v_ref, ks_ref, vs_ref):
    def rg(t):
        t = t.reshape(8, NR, 64, DH).transpose(1, 0, 2, 3)
        return t.reshape(1, NR, KR, DH)

    ks_ref[...] = rg(k_ref[0, :, 0, :])
    vs_ref[...] = rg(v_ref[0, :, 0, :])


def _a2a_body(ks_ref, vs_ref, km_ref, vm_ref, send_sems, recv_sems, loc_sems):
    my = lax.axis_index("i")

    barrier = pltpu.get_barrier_semaphore()
    for p in range(1, N_DEV):
        pl.semaphore_signal(
            barrier, inc=1,
            device_id=((my + p) % N_DEV,),
            device_id_type=pl.DeviceIdType.MESH,
        )
    pl.semaphore_wait(barrier, N_DEV - 1)

    kloc = pltpu.make_async_copy(
        ks_ref.at[pl.ds(my * HL, HL)], km_ref.at[my], loc_sems.at[0])
    vloc = pltpu.make_async_copy(
        vs_ref.at[pl.ds(my * HL, HL)], vm_ref.at[my], loc_sems.at[1])
    kloc.start()
    vloc.start()

    copies = []
    for p in range(1, N_DEV):
        peer = (my + p) % N_DEV
        for t, (s_ref, d_ref) in enumerate(((ks_ref, km_ref), (vs_ref, vm_ref))):
            rdma = pltpu.make_async_remote_copy(
                src_ref=s_ref.at[pl.ds(peer * HL, HL)],
                dst_ref=d_ref.at[my],
                send_sem=send_sems.at[2 * (p - 1) + t],
                recv_sem=recv_sems.at[2 * (p - 1) + t],
                device_id=(peer,),
                device_id_type=pl.DeviceIdType.MESH,
            )
            rdma.start()
            copies.append(rdma)
    for rdma in copies:
        rdma.wait()
    kloc.wait()
    vloc.wait()


def _attn_body(q_ref, k_ref, v_ref, o_ref):
    q = q_ref[0, 0]
    k = k_ref[:, 0, 0].reshape(N_DEV * KR, DH)
    v = v_ref[:, 0, 0].reshape(N_DEV * KR, DH)
    s = lax.dot_general(
        q, k, (((1,), (1,)), ((), ())), preferred_element_type=F32
    ) * SCALE
    m = jnp.max(s, axis=1, keepdims=True)
    w = jnp.exp(s - m)
    w = w / jnp.sum(w, axis=1, keepdims=True)
    o_ref[0, 0] = jnp.dot(w, v, preferred_element_type=F32)


def _oproj_body(o_ref, wo_ref, p_ref):
    h = pl.program_id(0)
    ctx = o_ref[0].reshape(NR, 8, 64, DH).transpose(1, 0, 2, 3).reshape(SQ, DH)
    ph = jnp.dot(ctx, wo_ref[...], preferred_element_type=F32)

    @pl.when(h == 0)
    def _():
        p_ref[...] = ph

    @pl.when(h != 0)
    def _():
        p_ref[...] += ph


def _allreduce_body(p_ref, out_ref, comm_ref, send_sems, recv_sems):
    my = lax.axis_index("i")
    left = (my - 1) % N_DEV
    right = (my + 1) % N_DEV

    barrier = pltpu.get_barrier_semaphore()
    for nbr in (left, right):
        pl.semaphore_signal(
            barrier, inc=1,
            device_id=(nbr,), device_id_type=pl.DeviceIdType.MESH,
        )
    pl.semaphore_wait(barrier, 2)

    part = p_ref[...]
    comm_ref[0] = part
    acc = part
    for h in range(N_DEV - 1):
        send_slot, recv_slot = h % 2, (h + 1) % 2
        rdma = pltpu.make_async_remote_copy(
            src_ref=comm_ref.at[send_slot],
            dst_ref=comm_ref.at[recv_slot],
            send_sem=send_sems.at[h],
            recv_sem=recv_sems.at[h],
            device_id=(right,),
            device_id_type=pl.DeviceIdType.MESH,
        )
        rdma.start()
        rdma.wait()
        acc = acc + comm_ref[recv_slot]
    out_ref[0] = acc


def kernel(x, Wq, K_ext, V_ext, Wo):
    q = pl.pallas_call(
        _qproj_body,
        grid=(HL,),
        in_specs=[
            pl.BlockSpec((1, SQ, DM), lambda h: (0, 0, 0)),
            pl.BlockSpec((DM, DH), lambda h: (0, h)),
        ],
        out_specs=pl.BlockSpec((1, NR, QR, DH), lambda h: (h, 0, 0, 0)),
        out_shape=jax.ShapeDtypeStruct((HL, NR, QR, DH), F32),
    )(x, Wq)

    ks, vs = pl.pallas_call(
        _regroup_body,
        grid=(HT,),
        in_specs=[
            pl.BlockSpec((1, SKV, 1, DH), lambda h: (0, 0, h, 0)),
            pl.BlockSpec((1, SKV, 1, DH), lambda h: (0, 0, h, 0)),
        ],
        out_specs=[
            pl.BlockSpec((1, NR, KR, DH), lambda h: (h, 0, 0, 0)),
            pl.BlockSpec((1, NR, KR, DH), lambda h: (h, 0, 0, 0)),
        ],
        out_shape=[jax.ShapeDtypeStruct((HT, NR, KR, DH), F32)] * 2,
    )(K_ext, V_ext)

    km, vm = pl.pallas_call(
        _a2a_body,
        in_specs=[pl.BlockSpec(memory_space=pl.ANY)] * 2,
        out_specs=[pl.BlockSpec(memory_space=pl.ANY)] * 2,
        out_shape=[jax.ShapeDtypeStruct((N_DEV, HL, NR, KR, DH), F32)] * 2,
        scratch_shapes=[
            pltpu.SemaphoreType.DMA((2 * (N_DEV - 1),)),
            pltpu.SemaphoreType.DMA((2 * (N_DEV - 1),)),
            pltpu.SemaphoreType.DMA((2,)),
        ],
        compiler_params=pltpu.CompilerParams(collective_id=0),
    )(ks, vs)

    o = pl.pallas_call(
        _attn_body,
        grid=(HL, NR),
        in_specs=[
            pl.BlockSpec((1, 1, QR, DH), lambda h, r: (h, r, 0, 0)),
            pl.BlockSpec((N_DEV, 1, 1, KR, DH), lambda h, r: (0, h, r, 0, 0)),
            pl.BlockSpec((N_DEV, 1, 1, KR, DH), lambda h, r: (0, h, r, 0, 0)),
        ],
        out_specs=pl.BlockSpec((1, 1, QR, DH), lambda h, r: (h, r, 0, 0)),
        out_shape=jax.ShapeDtypeStruct((HL, NR, QR, DH), F32),
    )(q, km, vm)

    part = pl.pallas_call(
        _oproj_body,
        grid=(HL,),
        in_specs=[
            pl.BlockSpec((1, NR, QR, DH), lambda h: (h, 0, 0, 0)),
            pl.BlockSpec((DH, DM), lambda h: (h, 0)),
        ],
        out_specs=pl.BlockSpec((SQ, DM), lambda h: (0, 0)),
        out_shape=jax.ShapeDtypeStruct((SQ, DM), F32),
    )(o, Wo)

    out = pl.pallas_call(
        _allreduce_body,
        in_specs=[pl.BlockSpec(memory_space=pltpu.VMEM)],
        out_specs=pl.BlockSpec(memory_space=pltpu.VMEM),
        out_shape=jax.ShapeDtypeStruct((1, SQ, DM), F32),
        scratch_shapes=[
            pltpu.VMEM((2, SQ, DM), F32),
            pltpu.SemaphoreType.DMA((N_DEV - 1,)),
            pltpu.SemaphoreType.DMA((N_DEV - 1,)),
        ],
        compiler_params=pltpu.CompilerParams(collective_id=1),
    )(part)
    return out


# baseline (device time: 1054291 ns/iter reference)
import jax
import jax.numpy as jnp
from jax import lax
from jax.experimental import pallas as pl
from jax.experimental.pallas import tpu as pltpu

N_DEV = 4
SQ = 2048
SKV = 2048
HL = 8
HT = 32
DH = 128
NR = 4
QR = SQ // NR
KR = SKV // NR
DM = 1024
SCALE = 0.08838834764831843
F32 = jnp.float32


def _qproj_body(x_ref, wq_ref, q_ref):
    qh = jnp.dot(x_ref[0], wq_ref[...], preferred_element_type=F32)
    qh = qh.reshape(8, NR, 64, DH).transpose(1, 0, 2, 3)
    q_ref[...] = qh.reshape(1, NR, QR, DH)


def _regroup_body(k_ref, v_ref, ks_ref, vs_ref):
    def rg(t):
        t = t.reshape(8, NR, 64, DH).transpose(1, 0, 2, 3)
        return t.reshape(1, NR, KR, DH)

    ks_ref[...] = rg(k_ref[...])
    vs_ref[...] = rg(v_ref[...])


def _a2a_body(ks_ref, vs_ref, km_ref, vm_ref, send_sems, recv_sems, loc_sems):
    my = lax.axis_index("i")

    barrier = pltpu.get_barrier_semaphore()
    for p in range(1, N_DEV):
        pl.semaphore_signal(
            barrier, inc=1,
            device_id=((my + p) % N_DEV,),
            device_id_type=pl.DeviceIdType.MESH,
        )
    pl.semaphore_wait(barrier, N_DEV - 1)

    kloc = pltpu.make_async_copy(
        ks_ref.at[pl.ds(my * HL, HL)], km_ref.at[my], loc_sems.at[0])
    vloc = pltpu.make_async_copy(
        vs_ref.at[pl.ds(my * HL, HL)], vm_ref.at[my], loc_sems.at[1])
    kloc.start()
    vloc.start()

    copies = []
    for p in range(1, N_DEV):
        peer = (my + p) % N_DEV
        for t, (s_ref, d_ref) in enumerate(((ks_ref, km_ref), (vs_ref, vm_ref))):
            rdma = pltpu.make_async_remote_copy(
                src_ref=s_ref.at[pl.ds(peer * HL, HL)],
                dst_ref=d_ref.at[my],
                send_sem=send_sems.at[2 * (p - 1) + t],
                recv_sem=recv_sems.at[2 * (p - 1) + t],
                device_id=(peer,),
                device_id_type=pl.DeviceIdType.MESH,
            )
            rdma.start()
            copies.append(rdma)
    for rdma in copies:
        rdma.wait()
    kloc.wait()
    vloc.wait()


def _attn_body(q_ref, k_ref, v_ref, o_ref):
    q = q_ref[0, 0]
    k = k_ref[:, 0, 0].reshape(N_DEV * KR, DH)
    v = v_ref[:, 0, 0].reshape(N_DEV * KR, DH)
    s = lax.dot_general(
        q, k, (((1,), (1,)), ((), ())), preferred_element_type=F32
    ) * SCALE
    m = jnp.max(s, axis=1, keepdims=True)
    w = jnp.exp(s - m)
    w = w / jnp.sum(w, axis=1, keepdims=True)
    o_ref[0, 0] = jnp.dot(w, v, preferred_element_type=F32)


def _oproj_body(o_ref, wo_ref, p_ref):
    h = pl.program_id(0)
    ctx = o_ref[0].reshape(NR, 8, 64, DH).transpose(1, 0, 2, 3).reshape(SQ, DH)
    ph = jnp.dot(ctx, wo_ref[...], preferred_element_type=F32)

    @pl.when(h == 0)
    def _():
        p_ref[...] = ph

    @pl.when(h != 0)
    def _():
        p_ref[...] += ph


def _allreduce_body(p_ref, out_ref, comm_ref, send_sems, recv_sems):
    my = lax.axis_index("i")
    left = (my - 1) % N_DEV
    right = (my + 1) % N_DEV

    barrier = pltpu.get_barrier_semaphore()
    for nbr in (left, right):
        pl.semaphore_signal(
            barrier, inc=1,
            device_id=(nbr,), device_id_type=pl.DeviceIdType.MESH,
        )
    pl.semaphore_wait(barrier, 2)

    part = p_ref[...]
    comm_ref[0] = part
    acc = part
    for h in range(N_DEV - 1):
        send_slot, recv_slot = h % 2, (h + 1) % 2
        rdma = pltpu.make_async_remote_copy(
            src_ref=comm_ref.at[send_slot],
            dst_ref=comm_ref.at[recv_slot],
            send_sem=send_sems.at[h],
            recv_sem=recv_sems.at[h],
            device_id=(right,),
            device_id_type=pl.DeviceIdType.MESH,
        )
        rdma.start()
        rdma.wait()
        acc = acc + comm_ref[recv_slot]
    out_ref[0] = acc


def kernel(x, Wq, K_ext, V_ext, Wo):
    q = pl.pallas_call(
        _qproj_body,
        grid=(HL,),
        in_specs=[
            pl.BlockSpec((1, SQ, DM), lambda h: (0, 0, 0)),
            pl.BlockSpec((DM, DH), lambda h: (0, h)),
        ],
        out_specs=pl.BlockSpec((1, NR, QR, DH), lambda h: (h, 0, 0, 0)),
        out_shape=jax.ShapeDtypeStruct((HL, NR, QR, DH), F32),
    )(x, Wq)

    ks, vs = pl.pallas_call(
        _regroup_body,
        grid=(HT,),
        in_specs=[
            pl.BlockSpec((SKV, DH), lambda h: (0, h)),
            pl.BlockSpec((SKV, DH), lambda h: (0, h)),
        ],
        out_specs=[
            pl.BlockSpec((1, NR, KR, DH), lambda h: (h, 0, 0, 0)),
            pl.BlockSpec((1, NR, KR, DH), lambda h: (h, 0, 0, 0)),
        ],
        out_shape=[jax.ShapeDtypeStruct((HT, NR, KR, DH), F32)] * 2,
    )(K_ext.reshape(SKV, HT * DH), V_ext.reshape(SKV, HT * DH))

    km, vm = pl.pallas_call(
        _a2a_body,
        in_specs=[pl.BlockSpec(memory_space=pl.ANY)] * 2,
        out_specs=[pl.BlockSpec(memory_space=pl.ANY)] * 2,
        out_shape=[jax.ShapeDtypeStruct((N_DEV, HL, NR, KR, DH), F32)] * 2,
        scratch_shapes=[
            pltpu.SemaphoreType.DMA((2 * (N_DEV - 1),)),
            pltpu.SemaphoreType.DMA((2 * (N_DEV - 1),)),
            pltpu.SemaphoreType.DMA((2,)),
        ],
        compiler_params=pltpu.CompilerParams(collective_id=0),
    )(ks, vs)

    o = pl.pallas_call(
        _attn_body,
        grid=(HL, NR),
        in_specs=[
            pl.BlockSpec((1, 1, QR, DH), lambda h, r: (h, r, 0, 0)),
            pl.BlockSpec((N_DEV, 1, 1, KR, DH), lambda h, r: (0, h, r, 0, 0)),
            pl.BlockSpec((N_DEV, 1, 1, KR, DH), lambda h, r: (0, h, r, 0, 0)),
        ],
        out_specs=pl.BlockSpec((1, 1, QR, DH), lambda h, r: (h, r, 0, 0)),
        out_shape=jax.ShapeDtypeStruct((HL, NR, QR, DH), F32),
    )(q, km, vm)

    part = pl.pallas_call(
        _oproj_body,
        grid=(HL,),
        in_specs=[
            pl.BlockSpec((1, NR, QR, DH), lambda h: (h, 0, 0, 0)),
            pl.BlockSpec((DH, DM), lambda h: (h, 0)),
        ],
        out_specs=pl.BlockSpec((SQ, DM), lambda h: (0, 0)),
        out_shape=jax.ShapeDtypeStruct((SQ, DM), F32),
    )(o, Wo)

    out = pl.pallas_call(
        _allreduce_body,
        in_specs=[pl.BlockSpec(memory_space=pltpu.VMEM)],
        out_specs=pl.BlockSpec(memory_space=pltpu.VMEM),
        out_shape=jax.ShapeDtypeStruct((1, SQ, DM), F32),
        scratch_shapes=[
            pltpu.VMEM((2, SQ, DM), F32),
            pltpu.SemaphoreType.DMA((N_DEV - 1,)),
            pltpu.SemaphoreType.DMA((N_DEV - 1,)),
        ],
        compiler_params=pltpu.CompilerParams(collective_id=1),
    )(part)
    return out


# device time: 603080 ns/iter; 1.7482x vs baseline; 1.7482x over previous
import jax
import jax.numpy as jnp
from jax import lax
from jax.experimental import pallas as pl
from jax.experimental.pallas import tpu as pltpu

N_DEV = 4
SQ = 2048
SKV = 2048
HL = 8
HT = 32
DH = 128
NR = 4
QR = SQ // NR
KR = SKV // NR
DM = 1024
SCALE = 0.08838834764831843
F32 = jnp.float32
BF16 = jnp.bfloat16


def _qproj_body(x_ref, wq_ref, q_ref):
    qh = jnp.dot(x_ref[0], wq_ref[...], preferred_element_type=F32)
    qh = qh.reshape(8, NR, 64, DH).transpose(1, 0, 2, 3)
    q_ref[...] = qh.reshape(1, NR, QR, DH)


def _regroup_body(k_ref, v_ref, ks_ref, vs_ref):
    def rg(t):
        t = t.reshape(8, NR, 64, DH).transpose(1, 0, 2, 3)
        return t.reshape(1, NR, KR, DH).astype(BF16)

    ks_ref[...] = rg(k_ref[...])
    vs_ref[...] = rg(v_ref[...])


def _a2a_body(ks_ref, vs_ref, km_ref, vm_ref, send_sems, recv_sems, loc_sems):
    my = lax.axis_index("i")

    barrier = pltpu.get_barrier_semaphore()
    for p in range(1, N_DEV):
        pl.semaphore_signal(
            barrier, inc=1,
            device_id=((my + p) % N_DEV,),
            device_id_type=pl.DeviceIdType.MESH,
        )
    pl.semaphore_wait(barrier, N_DEV - 1)

    kloc = pltpu.make_async_copy(
        ks_ref.at[pl.ds(my * HL, HL)], km_ref.at[my], loc_sems.at[0])
    vloc = pltpu.make_async_copy(
        vs_ref.at[pl.ds(my * HL, HL)], vm_ref.at[my], loc_sems.at[1])
    kloc.start()
    vloc.start()

    copies = []
    for p in range(1, N_DEV):
        peer = (my + p) % N_DEV
        for t, (s_ref, d_ref) in enumerate(((ks_ref, km_ref), (vs_ref, vm_ref))):
            rdma = pltpu.make_async_remote_copy(
                src_ref=s_ref.at[pl.ds(peer * HL, HL)],
                dst_ref=d_ref.at[my],
                send_sem=send_sems.at[2 * (p - 1) + t],
                recv_sem=recv_sems.at[2 * (p - 1) + t],
                device_id=(peer,),
                device_id_type=pl.DeviceIdType.MESH,
            )
            rdma.start()
            copies.append(rdma)
    for rdma in copies:
        rdma.wait()
    kloc.wait()
    vloc.wait()


def _attn_body(q_ref, k_ref, v_ref, o_ref):
    q = q_ref[0, 0].astype(BF16)
    k = k_ref[:, 0, 0].reshape(N_DEV * KR, DH)
    v = v_ref[:, 0, 0].reshape(N_DEV * KR, DH)
    s = lax.dot_general(
        q, k, (((1,), (1,)), ((), ())), preferred_element_type=F32
    ) * SCALE
    m = jnp.max(s, axis=1, keepdims=True)
    w = jnp.exp(s - m)
    w = (w / jnp.sum(w, axis=1, keepdims=True)).astype(BF16)
    o_ref[0, 0] = jnp.dot(w, v, preferred_element_type=F32)


def _oproj_body(o_ref, wo_ref, p_ref):
    h = pl.program_id(0)
    ctx = o_ref[0].reshape(NR, 8, 64, DH).transpose(1, 0, 2, 3).reshape(SQ, DH)
    ph = jnp.dot(ctx, wo_ref[...], preferred_element_type=F32)

    @pl.when(h == 0)
    def _():
        p_ref[...] = ph

    @pl.when(h != 0)
    def _():
        p_ref[...] += ph


CH = SQ // N_DEV


def _allreduce_body(p_ref, out_ref, comm_ref, send_sems, recv_sems):
    my = lax.axis_index("i")
    left = (my - 1) % N_DEV
    right = (my + 1) % N_DEV

    barrier = pltpu.get_barrier_semaphore()
    for nbr in (left, right):
        pl.semaphore_signal(
            barrier, inc=1,
            device_id=(nbr,), device_id_type=pl.DeviceIdType.MESH,
        )
    pl.semaphore_wait(barrier, 2)

    def hop(u):
        rdma = pltpu.make_async_remote_copy(
            src_ref=comm_ref.at[u % 2],
            dst_ref=comm_ref.at[(u + 1) % 2],
            send_sem=send_sems.at[u],
            recv_sem=recv_sems.at[u],
            device_id=(right,),
            device_id_type=pl.DeviceIdType.MESH,
        )
        rdma.start()
        rdma.wait()

    def chunk(c):
        return p_ref[pl.ds(c * CH, CH), :]

    comm_ref[0] = chunk((my - 1) % N_DEV)
    hop(0)
    comm_ref[1] += chunk((my - 2) % N_DEV)
    hop(1)
    comm_ref[0] += chunk((my - 3) % N_DEV)
    hop(2)
    comm_ref[1] += chunk(my)
    out_ref[0, pl.ds(my * CH, CH), :] = comm_ref[1]

    for t in range(N_DEV - 1):
        hop(3 + t)
        c = (my - 1 - t) % N_DEV
        out_ref[0, pl.ds(c * CH, CH), :] = comm_ref[t % 2]


def kernel(x, Wq, K_ext, V_ext, Wo):
    q = pl.pallas_call(
        _qproj_body,
        grid=(HL,),
        in_specs=[
            pl.BlockSpec((1, SQ, DM), lambda h: (0, 0, 0)),
            pl.BlockSpec((DM, DH), lambda h: (0, h)),
        ],
        out_specs=pl.BlockSpec((1, NR, QR, DH), lambda h: (h, 0, 0, 0)),
        out_shape=jax.ShapeDtypeStruct((HL, NR, QR, DH), F32),
    )(x, Wq)

    ks, vs = pl.pallas_call(
        _regroup_body,
        grid=(HT,),
        in_specs=[
            pl.BlockSpec((SKV, DH), lambda h: (0, h)),
            pl.BlockSpec((SKV, DH), lambda h: (0, h)),
        ],
        out_specs=[
            pl.BlockSpec((1, NR, KR, DH), lambda h: (h, 0, 0, 0)),
            pl.BlockSpec((1, NR, KR, DH), lambda h: (h, 0, 0, 0)),
        ],
        out_shape=[jax.ShapeDtypeStruct((HT, NR, KR, DH), BF16)] * 2,
    )(K_ext.reshape(SKV, HT * DH), V_ext.reshape(SKV, HT * DH))

    km, vm = pl.pallas_call(
        _a2a_body,
        in_specs=[pl.BlockSpec(memory_space=pl.ANY)] * 2,
        out_specs=[pl.BlockSpec(memory_space=pl.ANY)] * 2,
        out_shape=[jax.ShapeDtypeStruct((N_DEV, HL, NR, KR, DH), BF16)] * 2,
        scratch_shapes=[
            pltpu.SemaphoreType.DMA((2 * (N_DEV - 1),)),
            pltpu.SemaphoreType.DMA((2 * (N_DEV - 1),)),
            pltpu.SemaphoreType.DMA((2,)),
        ],
        compiler_params=pltpu.CompilerParams(collective_id=0),
    )(ks, vs)

    o = pl.pallas_call(
        _attn_body,
        grid=(HL, NR),
        in_specs=[
            pl.BlockSpec((1, 1, QR, DH), lambda h, r: (h, r, 0, 0)),
            pl.BlockSpec((N_DEV, 1, 1, KR, DH), lambda h, r: (0, h, r, 0, 0)),
            pl.BlockSpec((N_DEV, 1, 1, KR, DH), lambda h, r: (0, h, r, 0, 0)),
        ],
        out_specs=pl.BlockSpec((1, 1, QR, DH), lambda h, r: (h, r, 0, 0)),
        out_shape=jax.ShapeDtypeStruct((HL, NR, QR, DH), F32),
    )(q, km, vm)

    part = pl.pallas_call(
        _oproj_body,
        grid=(HL,),
        in_specs=[
            pl.BlockSpec((1, NR, QR, DH), lambda h: (h, 0, 0, 0)),
            pl.BlockSpec((DH, DM), lambda h: (h, 0)),
        ],
        out_specs=pl.BlockSpec((SQ, DM), lambda h: (0, 0)),
        out_shape=jax.ShapeDtypeStruct((SQ, DM), F32),
    )(o, Wo)

    out = pl.pallas_call(
        _allreduce_body,
        in_specs=[pl.BlockSpec(memory_space=pltpu.VMEM)],
        out_specs=pl.BlockSpec(memory_space=pltpu.VMEM),
        out_shape=jax.ShapeDtypeStruct((1, SQ, DM), F32),
        scratch_shapes=[
            pltpu.VMEM((2, CH, DM), F32),
            pltpu.SemaphoreType.DMA((2 * (N_DEV - 1),)),
            pltpu.SemaphoreType.DMA((2 * (N_DEV - 1),)),
        ],
        compiler_params=pltpu.CompilerParams(collective_id=1),
    )(part)
    return out


# device time: 462814 ns/iter; 2.2780x vs baseline; 1.3031x over previous
import jax
import jax.numpy as jnp
from jax import lax
from jax.experimental import pallas as pl
from jax.experimental.pallas import tpu as pltpu

N_DEV = 4
SQ = 2048
SKV = 2048
HL = 8
HT = 32
DH = 128
NR = 4
QR = SQ // NR
KR = SKV // NR
DM = 1024
SCALE = 0.08838834764831843
F32 = jnp.float32
BF16 = jnp.bfloat16


def _qproj_body(x_ref, wq_ref, q_ref):
    qh = jnp.dot(x_ref[0], wq_ref[...], preferred_element_type=F32)
    qh = qh.reshape(8, NR, 64, DH).transpose(1, 0, 2, 3)
    q_ref[...] = qh.reshape(1, NR, QR, DH)


def _regroup_body(k_ref, v_ref, ks_ref, vs_ref):
    def rg(t):
        t = t.reshape(8, NR, 64, 8, DH).transpose(3, 1, 0, 2, 4)
        return t.reshape(8, NR, KR, DH).astype(BF16)

    ks_ref[...] = rg(k_ref[0])
    vs_ref[...] = rg(v_ref[0])


def _a2a_body(ks_ref, vs_ref, km_ref, vm_ref, send_sems, recv_sems, loc_sems):
    my = lax.axis_index("i")

    barrier = pltpu.get_barrier_semaphore()
    for p in range(1, N_DEV):
        pl.semaphore_signal(
            barrier, inc=1,
            device_id=((my + p) % N_DEV,),
            device_id_type=pl.DeviceIdType.MESH,
        )
    pl.semaphore_wait(barrier, N_DEV - 1)

    kloc = pltpu.make_async_copy(
        ks_ref.at[pl.ds(my * HL, HL)], km_ref.at[my], loc_sems.at[0])
    vloc = pltpu.make_async_copy(
        vs_ref.at[pl.ds(my * HL, HL)], vm_ref.at[my], loc_sems.at[1])
    kloc.start()
    vloc.start()

    copies = []
    for p in range(1, N_DEV):
        peer = (my + p) % N_DEV
        for t, (s_ref, d_ref) in enumerate(((ks_ref, km_ref), (vs_ref, vm_ref))):
            rdma = pltpu.make_async_remote_copy(
                src_ref=s_ref.at[pl.ds(peer * HL, HL)],
                dst_ref=d_ref.at[my],
                send_sem=send_sems.at[2 * (p - 1) + t],
                recv_sem=recv_sems.at[2 * (p - 1) + t],
                device_id=(peer,),
                device_id_type=pl.DeviceIdType.MESH,
            )
            rdma.start()
            copies.append(rdma)
    for rdma in copies:
        rdma.wait()
    kloc.wait()
    vloc.wait()


def _attn_body(q_ref, k_ref, v_ref, o_ref):
    q = q_ref[0, 0].astype(BF16)
    k = k_ref[:, 0, 0].reshape(N_DEV * KR, DH)
    v = v_ref[:, 0, 0].reshape(N_DEV * KR, DH)
    s = lax.dot_general(
        q, k, (((1,), (1,)), ((), ())), preferred_element_type=F32
    ) * SCALE
    w = jnp.exp(s)
    o_ref[0, 0] = jnp.dot(
        w.astype(BF16), v, preferred_element_type=F32
    ) / jnp.sum(w, axis=1, keepdims=True)


def _oproj_body(o_ref, wo_ref, p_ref):
    h = pl.program_id(0)
    ctx = o_ref[0].reshape(NR, 8, 64, DH).transpose(1, 0, 2, 3).reshape(SQ, DH)
    ph = jnp.dot(ctx, wo_ref[...], preferred_element_type=F32)

    @pl.when(h == 0)
    def _():
        p_ref[...] = ph

    @pl.when(h != 0)
    def _():
        p_ref[...] += ph


CH = SQ // N_DEV


def _allreduce_body(p_ref, out_ref, comm_ref, send_sems, recv_sems):
    my = lax.axis_index("i")
    left = (my - 1) % N_DEV
    right = (my + 1) % N_DEV

    barrier = pltpu.get_barrier_semaphore()
    for nbr in (left, right):
        pl.semaphore_signal(
            barrier, inc=1,
            device_id=(nbr,), device_id_type=pl.DeviceIdType.MESH,
        )
    pl.semaphore_wait(barrier, 2)

    def hop(u):
        rdma = pltpu.make_async_remote_copy(
            src_ref=comm_ref.at[u % 2],
            dst_ref=comm_ref.at[(u + 1) % 2],
            send_sem=send_sems.at[u],
            recv_sem=recv_sems.at[u],
            device_id=(right,),
            device_id_type=pl.DeviceIdType.MESH,
        )
        rdma.start()
        rdma.wait()

    def chunk(c):
        return p_ref[pl.ds(c * CH, CH), :]

    comm_ref[0] = chunk((my - 1) % N_DEV)
    hop(0)
    comm_ref[1] += chunk((my - 2) % N_DEV)
    hop(1)
    comm_ref[0] += chunk((my - 3) % N_DEV)
    hop(2)
    comm_ref[1] += chunk(my)
    out_ref[0, pl.ds(my * CH, CH), :] = comm_ref[1]

    for t in range(N_DEV - 1):
        hop(3 + t)
        c = (my - 1 - t) % N_DEV
        out_ref[0, pl.ds(c * CH, CH), :] = comm_ref[t % 2]


def kernel(x, Wq, K_ext, V_ext, Wo):
    q = pl.pallas_call(
        _qproj_body,
        grid=(HL,),
        in_specs=[
            pl.BlockSpec((1, SQ, DM), lambda h: (0, 0, 0)),
            pl.BlockSpec((DM, DH), lambda h: (0, h)),
        ],
        out_specs=pl.BlockSpec((1, NR, QR, DH), lambda h: (h, 0, 0, 0)),
        out_shape=jax.ShapeDtypeStruct((HL, NR, QR, DH), F32),
    )(x, Wq)

    ks, vs = pl.pallas_call(
        _regroup_body,
        grid=(N_DEV,),
        in_specs=[
            pl.BlockSpec((1, SKV, HL, DH), lambda g: (0, 0, g, 0)),
            pl.BlockSpec((1, SKV, HL, DH), lambda g: (0, 0, g, 0)),
        ],
        out_specs=[
            pl.BlockSpec((HL, NR, KR, DH), lambda g: (g, 0, 0, 0)),
            pl.BlockSpec((HL, NR, KR, DH), lambda g: (g, 0, 0, 0)),
        ],
        out_shape=[jax.ShapeDtypeStruct((HT, NR, KR, DH), BF16)] * 2,
        compiler_params=pltpu.CompilerParams(
            vmem_limit_bytes=100 * 1024 * 1024
        ),
    )(K_ext, V_ext)

    km, vm = pl.pallas_call(
        _a2a_body,
        in_specs=[pl.BlockSpec(memory_space=pl.ANY)] * 2,
        out_specs=[pl.BlockSpec(memory_space=pl.ANY)] * 2,
        out_shape=[jax.ShapeDtypeStruct((N_DEV, HL, NR, KR, DH), BF16)] * 2,
        scratch_shapes=[
            pltpu.SemaphoreType.DMA((2 * (N_DEV - 1),)),
            pltpu.SemaphoreType.DMA((2 * (N_DEV - 1),)),
            pltpu.SemaphoreType.DMA((2,)),
        ],
        compiler_params=pltpu.CompilerParams(collective_id=0),
    )(ks, vs)

    o = pl.pallas_call(
        _attn_body,
        grid=(HL, NR),
        in_specs=[
            pl.BlockSpec((1, 1, QR, DH), lambda h, r: (h, r, 0, 0)),
            pl.BlockSpec((N_DEV, 1, 1, KR, DH), lambda h, r: (0, h, r, 0, 0)),
            pl.BlockSpec((N_DEV, 1, 1, KR, DH), lambda h, r: (0, h, r, 0, 0)),
        ],
        out_specs=pl.BlockSpec((1, 1, QR, DH), lambda h, r: (h, r, 0, 0)),
        out_shape=jax.ShapeDtypeStruct((HL, NR, QR, DH), F32),
    )(q, km, vm)

    part = pl.pallas_call(
        _oproj_body,
        grid=(HL,),
        in_specs=[
            pl.BlockSpec((1, NR, QR, DH), lambda h: (h, 0, 0, 0)),
            pl.BlockSpec((DH, DM), lambda h: (h, 0)),
        ],
        out_specs=pl.BlockSpec((SQ, DM), lambda h: (0, 0)),
        out_shape=jax.ShapeDtypeStruct((SQ, DM), F32),
    )(o, Wo)

    out = pl.pallas_call(
        _allreduce_body,
        in_specs=[pl.BlockSpec(memory_space=pltpu.VMEM)],
        out_specs=pl.BlockSpec(memory_space=pltpu.VMEM),
        out_shape=jax.ShapeDtypeStruct((1, SQ, DM), F32),
        scratch_shapes=[
            pltpu.VMEM((2, CH, DM), F32),
            pltpu.SemaphoreType.DMA((2 * (N_DEV - 1),)),
            pltpu.SemaphoreType.DMA((2 * (N_DEV - 1),)),
        ],
        compiler_params=pltpu.CompilerParams(collective_id=1),
    )(part)
    return out


# device time: 395001 ns/iter; 2.6691x vs baseline; 1.1717x over previous
import jax
import jax.numpy as jnp
from jax import lax
from jax.experimental import pallas as pl
from jax.experimental.pallas import tpu as pltpu

N_DEV = 4
SQ = 2048
SKV = 2048
HL = 8
HT = 32
DH = 128
NR = 4
QR = SQ // NR
KR = SKV // NR
DM = 1024
SCALE = 0.08838834764831843
F32 = jnp.float32
BF16 = jnp.bfloat16


def _qproj_body(x_ref, wq_ref, q_ref):
    qh = jnp.dot(x_ref[0], wq_ref[...], preferred_element_type=F32)
    qh = qh.reshape(8, NR, 64, DH).transpose(1, 0, 2, 3)
    q_ref[...] = qh.reshape(1, NR, QR, DH)


def _regroup_body(k_ref, v_ref, ks_ref, vs_ref):
    def rg(t):
        t = t.reshape(8, NR, 64, 8, DH).transpose(3, 1, 0, 2, 4)
        return t.reshape(8, NR, KR, DH).astype(BF16)

    ks_ref[...] = rg(k_ref[0])
    vs_ref[...] = rg(v_ref[0])


def _a2a_body(ks_ref, vs_ref, km_ref, vm_ref, send_sems, recv_sems, loc_sems):
    my = lax.axis_index("i")

    barrier = pltpu.get_barrier_semaphore()
    for p in range(1, N_DEV):
        pl.semaphore_signal(
            barrier, inc=1,
            device_id=((my + p) % N_DEV,),
            device_id_type=pl.DeviceIdType.MESH,
        )
    pl.semaphore_wait(barrier, N_DEV - 1)

    kloc = pltpu.make_async_copy(
        ks_ref.at[pl.ds(my * HL, HL)], km_ref.at[my], loc_sems.at[0])
    vloc = pltpu.make_async_copy(
        vs_ref.at[pl.ds(my * HL, HL)], vm_ref.at[my], loc_sems.at[1])
    kloc.start()
    vloc.start()

    copies = []
    for p in range(1, N_DEV):
        peer = (my + p) % N_DEV
        for t, (s_ref, d_ref) in enumerate(((ks_ref, km_ref), (vs_ref, vm_ref))):
            rdma = pltpu.make_async_remote_copy(
                src_ref=s_ref.at[pl.ds(peer * HL, HL)],
                dst_ref=d_ref.at[my],
                send_sem=send_sems.at[2 * (p - 1) + t],
                recv_sem=recv_sems.at[2 * (p - 1) + t],
                device_id=(peer,),
                device_id_type=pl.DeviceIdType.MESH,
            )
            rdma.start()
            copies.append(rdma)
    for rdma in copies:
        rdma.wait()
    kloc.wait()
    vloc.wait()


def _attn_body(q_ref, k_ref, v_ref, o_ref):
    q = q_ref[0, 0].astype(BF16)
    k = k_ref[:, 0, 0].reshape(N_DEV * KR, DH)
    v = v_ref[:, 0, 0].reshape(N_DEV * KR, DH)
    s = lax.dot_general(
        q, k, (((1,), (1,)), ((), ())), preferred_element_type=F32
    ) * SCALE
    w = jnp.exp(s)
    o_ref[0, 0] = jnp.dot(
        w.astype(BF16), v, preferred_element_type=F32
    ) / jnp.sum(w, axis=1, keepdims=True)


def _oproj_body(o_ref, wo_ref, p_ref):
    h = pl.program_id(0)
    ctx = o_ref[0].reshape(NR, 8, 64, DH).transpose(1, 0, 2, 3).reshape(SQ, DH)
    ph = jnp.dot(ctx, wo_ref[...], preferred_element_type=F32)

    @pl.when(h == 0)
    def _():
        p_ref[...] = ph

    @pl.when(h != 0)
    def _():
        p_ref[...] += ph


CH = SQ // N_DEV


def _allreduce_body(p_ref, out_ref, comm_ref, send_sems, recv_sems):
    my = lax.axis_index("i")
    left = (my - 1) % N_DEV
    right = (my + 1) % N_DEV

    barrier = pltpu.get_barrier_semaphore()
    for nbr in (left, right):
        pl.semaphore_signal(
            barrier, inc=1,
            device_id=(nbr,), device_id_type=pl.DeviceIdType.MESH,
        )
    pl.semaphore_wait(barrier, 2)

    def hop(u):
        rdma = pltpu.make_async_remote_copy(
            src_ref=comm_ref.at[u % 2],
            dst_ref=comm_ref.at[(u + 1) % 2],
            send_sem=send_sems.at[u],
            recv_sem=recv_sems.at[u],
            device_id=(right,),
            device_id_type=pl.DeviceIdType.MESH,
        )
        rdma.start()
        rdma.wait()

    def chunk(c):
        return p_ref[pl.ds(c * CH, CH), :]

    def accum(slot, c):
        comm_ref[slot] = (comm_ref[slot].astype(F32) + chunk(c)).astype(BF16)

    comm_ref[0] = chunk((my - 1) % N_DEV).astype(BF16)
    hop(0)
    accum(1, (my - 2) % N_DEV)
    hop(1)
    accum(0, (my - 3) % N_DEV)
    hop(2)
    accum(1, my)
    out_ref[0, pl.ds(my * CH, CH), :] = comm_ref[1].astype(F32)

    for t in range(N_DEV - 1):
        hop(3 + t)
        c = (my - 1 - t) % N_DEV
        out_ref[0, pl.ds(c * CH, CH), :] = comm_ref[t % 2].astype(F32)


def kernel(x, Wq, K_ext, V_ext, Wo):
    q = pl.pallas_call(
        _qproj_body,
        grid=(HL,),
        in_specs=[
            pl.BlockSpec((1, SQ, DM), lambda h: (0, 0, 0)),
            pl.BlockSpec((DM, DH), lambda h: (0, h)),
        ],
        out_specs=pl.BlockSpec((1, NR, QR, DH), lambda h: (h, 0, 0, 0)),
        out_shape=jax.ShapeDtypeStruct((HL, NR, QR, DH), F32),
    )(x, Wq)

    ks, vs = pl.pallas_call(
        _regroup_body,
        grid=(N_DEV,),
        in_specs=[
            pl.BlockSpec((1, SKV, HL, DH), lambda g: (0, 0, g, 0)),
            pl.BlockSpec((1, SKV, HL, DH), lambda g: (0, 0, g, 0)),
        ],
        out_specs=[
            pl.BlockSpec((HL, NR, KR, DH), lambda g: (g, 0, 0, 0)),
            pl.BlockSpec((HL, NR, KR, DH), lambda g: (g, 0, 0, 0)),
        ],
        out_shape=[jax.ShapeDtypeStruct((HT, NR, KR, DH), BF16)] * 2,
        compiler_params=pltpu.CompilerParams(
            vmem_limit_bytes=100 * 1024 * 1024
        ),
    )(K_ext, V_ext)

    km, vm = pl.pallas_call(
        _a2a_body,
        in_specs=[pl.BlockSpec(memory_space=pl.ANY)] * 2,
        out_specs=[pl.BlockSpec(memory_space=pl.ANY)] * 2,
        out_shape=[jax.ShapeDtypeStruct((N_DEV, HL, NR, KR, DH), BF16)] * 2,
        scratch_shapes=[
            pltpu.SemaphoreType.DMA((2 * (N_DEV - 1),)),
            pltpu.SemaphoreType.DMA((2 * (N_DEV - 1),)),
            pltpu.SemaphoreType.DMA((2,)),
        ],
        compiler_params=pltpu.CompilerParams(collective_id=0),
    )(ks, vs)

    o = pl.pallas_call(
        _attn_body,
        grid=(HL, NR),
        in_specs=[
            pl.BlockSpec((1, 1, QR, DH), lambda h, r: (h, r, 0, 0)),
            pl.BlockSpec((N_DEV, 1, 1, KR, DH), lambda h, r: (0, h, r, 0, 0)),
            pl.BlockSpec((N_DEV, 1, 1, KR, DH), lambda h, r: (0, h, r, 0, 0)),
        ],
        out_specs=pl.BlockSpec((1, 1, QR, DH), lambda h, r: (h, r, 0, 0)),
        out_shape=jax.ShapeDtypeStruct((HL, NR, QR, DH), F32),
    )(q, km, vm)

    part = pl.pallas_call(
        _oproj_body,
        grid=(HL,),
        in_specs=[
            pl.BlockSpec((1, NR, QR, DH), lambda h: (h, 0, 0, 0)),
            pl.BlockSpec((DH, DM), lambda h: (h, 0)),
        ],
        out_specs=pl.BlockSpec((SQ, DM), lambda h: (0, 0)),
        out_shape=jax.ShapeDtypeStruct((SQ, DM), F32),
    )(o, Wo)

    out = pl.pallas_call(
        _allreduce_body,
        in_specs=[pl.BlockSpec(memory_space=pltpu.VMEM)],
        out_specs=pl.BlockSpec(memory_space=pltpu.VMEM),
        out_shape=jax.ShapeDtypeStruct((1, SQ, DM), F32),
        scratch_shapes=[
            pltpu.VMEM((2, CH, DM), BF16),
            pltpu.SemaphoreType.DMA((2 * (N_DEV - 1),)),
            pltpu.SemaphoreType.DMA((2 * (N_DEV - 1),)),
        ],
        compiler_params=pltpu.CompilerParams(collective_id=1),
    )(part)
    return out


# device time: 388064 ns/iter; 2.7168x vs baseline; 1.0179x over previous
import jax
import jax.numpy as jnp
from jax import lax
from jax.experimental import pallas as pl
from jax.experimental.pallas import tpu as pltpu

N_DEV = 4
SQ = 2048
SKV = 2048
HL = 8
HT = 32
DH = 128
NR = 4
QR = SQ // NR
KR = SKV // NR
DM = 1024
SCALE = 0.08838834764831843
F32 = jnp.float32
BF16 = jnp.bfloat16


def _qproj_body(x_ref, wq_ref, q_ref):
    qh = jnp.dot(x_ref[0], wq_ref[...], preferred_element_type=F32)
    qh = qh.reshape(8, NR, 64, DH).transpose(1, 0, 2, 3)
    q_ref[...] = qh.reshape(1, NR, QR, DH).astype(BF16)


def _fused_body(q_ref, ks_ref, vs_ref, o_ref,
                l_ref, km_ref, vm_ref, send_sems, recv_sems, loc_sems):
    my = lax.axis_index("i")

    barrier = pltpu.get_barrier_semaphore()
    for p in range(1, N_DEV):
        pl.semaphore_signal(
            barrier, inc=1,
            device_id=((my + p) % N_DEV,),
            device_id_type=pl.DeviceIdType.MESH,
        )
    pl.semaphore_wait(barrier, N_DEV - 1)

    kloc = pltpu.make_async_copy(
        ks_ref.at[pl.ds(my * HL, HL)], km_ref.at[my], loc_sems.at[0])
    vloc = pltpu.make_async_copy(
        vs_ref.at[pl.ds(my * HL, HL)], vm_ref.at[my], loc_sems.at[1])
    kloc.start()
    vloc.start()
    rdmas = []
    for p in range(1, N_DEV):
        peer = (my + p) % N_DEV
        for t, (s_ref, d_ref) in enumerate(((ks_ref, km_ref), (vs_ref, vm_ref))):
            rdma = pltpu.make_async_remote_copy(
                src_ref=s_ref.at[pl.ds(peer * HL, HL)],
                dst_ref=d_ref.at[my],
                send_sem=send_sems.at[2 * (p - 1) + t],
                recv_sem=recv_sems.at[2 * (p - 1) + t],
                device_id=(peer,),
                device_id_type=pl.DeviceIdType.MESH,
            )
            rdma.start()
            rdmas.append(rdma)

    o_ref[...] = jnp.zeros((HL, NR, QR, DH), F32)
    l_ref[...] = jnp.zeros((HL, NR, QR, 1), F32)

    def consume(j):
        def pair(idx, _):
            h = idx // NR
            r = idx % NR
            q = q_ref[h, r]
            k = km_ref[j, h, r]
            v = vm_ref[j, h, r]
            s = lax.dot_general(
                q, k, (((1,), (1,)), ((), ())), preferred_element_type=F32
            ) * SCALE
            e = jnp.exp(s)
            l_ref[h, r] = l_ref[h, r] + jnp.sum(e, axis=1, keepdims=True)
            o_ref[h, r] = o_ref[h, r] + jnp.dot(
                e.astype(BF16), v, preferred_element_type=F32)
            return 0

        lax.fori_loop(0, HL * NR, pair, 0)

    def wait_recv_from(p):
        j = (my - p) % N_DEV
        for t, (s_ref, d_ref) in enumerate(((ks_ref, km_ref), (vs_ref, vm_ref))):
            pltpu.make_async_remote_copy(
                src_ref=s_ref.at[pl.ds(0, HL)],
                dst_ref=d_ref.at[j],
                send_sem=send_sems.at[2 * (p - 1) + t],
                recv_sem=recv_sems.at[2 * (p - 1) + t],
                device_id=(my,),
                device_id_type=pl.DeviceIdType.MESH,
            ).wait_recv()
        return j

    kloc.wait()
    vloc.wait()
    consume(my)
    for p in (1, 3, 2):
        consume(wait_recv_from(p))

    def normalize(idx, _):
        h = idx // NR
        r = idx % NR
        o_ref[h, r] = o_ref[h, r] / l_ref[h, r]
        return 0

    lax.fori_loop(0, HL * NR, normalize, 0)

    for rdma in rdmas:
        rdma.wait_send()


def _regroup_body(k_ref, v_ref, ks_ref, vs_ref):
    def rg(t):
        t = t.reshape(8, NR, 64, 8, DH).transpose(3, 1, 0, 2, 4)
        return t.reshape(8, NR, KR, DH).astype(BF16)

    ks_ref[...] = rg(k_ref[0])
    vs_ref[...] = rg(v_ref[0])


def _oproj_body(o_ref, wo_ref, p_ref):
    h = pl.program_id(0)
    ctx = o_ref[0].reshape(NR, 8, 64, DH).transpose(1, 0, 2, 3).reshape(SQ, DH)
    ph = jnp.dot(ctx, wo_ref[...], preferred_element_type=F32)

    @pl.when(h == 0)
    def _():
        p_ref[...] = ph

    @pl.when(h != 0)
    def _():
        p_ref[...] += ph


CH = SQ // N_DEV


def _allreduce_body(p_ref, out_ref, comm_ref, send_sems, recv_sems):
    my = lax.axis_index("i")
    left = (my - 1) % N_DEV
    right = (my + 1) % N_DEV

    barrier = pltpu.get_barrier_semaphore()
    for nbr in (left, right):
        pl.semaphore_signal(
            barrier, inc=1,
            device_id=(nbr,), device_id_type=pl.DeviceIdType.MESH,
        )
    pl.semaphore_wait(barrier, 2)

    def hop(u):
        rdma = pltpu.make_async_remote_copy(
            src_ref=comm_ref.at[u % 2],
            dst_ref=comm_ref.at[(u + 1) % 2],
            send_sem=send_sems.at[u],
            recv_sem=recv_sems.at[u],
            device_id=(right,),
            device_id_type=pl.DeviceIdType.MESH,
        )
        rdma.start()
        rdma.wait()

    def chunk(c):
        return p_ref[pl.ds(c * CH, CH), :]

    def accum(slot, c):
        comm_ref[slot] = (comm_ref[slot].astype(F32) + chunk(c)).astype(BF16)

    comm_ref[0] = chunk((my - 1) % N_DEV).astype(BF16)
    hop(0)
    accum(1, (my - 2) % N_DEV)
    hop(1)
    accum(0, (my - 3) % N_DEV)
    hop(2)
    accum(1, my)
    out_ref[0, pl.ds(my * CH, CH), :] = comm_ref[1].astype(F32)

    for t in range(N_DEV - 1):
        hop(3 + t)
        c = (my - 1 - t) % N_DEV
        out_ref[0, pl.ds(c * CH, CH), :] = comm_ref[t % 2].astype(F32)


def kernel(x, Wq, K_ext, V_ext, Wo):
    ks, vs = pl.pallas_call(
        _regroup_body,
        grid=(N_DEV,),
        in_specs=[
            pl.BlockSpec((1, SKV, HL, DH), lambda g: (0, 0, g, 0)),
            pl.BlockSpec((1, SKV, HL, DH), lambda g: (0, 0, g, 0)),
        ],
        out_specs=[
            pl.BlockSpec((HL, NR, KR, DH), lambda g: (g, 0, 0, 0)),
            pl.BlockSpec((HL, NR, KR, DH), lambda g: (g, 0, 0, 0)),
        ],
        out_shape=[jax.ShapeDtypeStruct((HT, NR, KR, DH), BF16)] * 2,
        compiler_params=pltpu.CompilerParams(
            vmem_limit_bytes=100 * 1024 * 1024
        ),
    )(K_ext, V_ext)

    q = pl.pallas_call(
        _qproj_body,
        grid=(HL,),
        in_specs=[
            pl.BlockSpec((1, SQ, DM), lambda h: (0, 0, 0)),
            pl.BlockSpec((DM, DH), lambda h: (0, h)),
        ],
        out_specs=pl.BlockSpec((1, NR, QR, DH), lambda h: (h, 0, 0, 0)),
        out_shape=jax.ShapeDtypeStruct((HL, NR, QR, DH), BF16),
        compiler_params=pltpu.CompilerParams(
            vmem_limit_bytes=60 * 1024 * 1024
        ),
    )(x, Wq)

    o = pl.pallas_call(
        _fused_body,
        in_specs=[
            pl.BlockSpec(memory_space=pltpu.VMEM),
            pl.BlockSpec(memory_space=pl.ANY),
            pl.BlockSpec(memory_space=pl.ANY),
        ],
        out_specs=pl.BlockSpec(memory_space=pltpu.VMEM),
        out_shape=jax.ShapeDtypeStruct((HL, NR, QR, DH), F32),
        scratch_shapes=[
            pltpu.VMEM((HL, NR, QR, 1), F32),
            pltpu.VMEM((N_DEV, HL, NR, KR, DH), BF16),
            pltpu.VMEM((N_DEV, HL, NR, KR, DH), BF16),
            pltpu.SemaphoreType.DMA((2 * (N_DEV - 1),)),
            pltpu.SemaphoreType.DMA((2 * (N_DEV - 1),)),
            pltpu.SemaphoreType.DMA((2,)),
        ],
        compiler_params=pltpu.CompilerParams(
            collective_id=0, vmem_limit_bytes=60 * 1024 * 1024
        ),
    )(q, ks, vs)

    part = pl.pallas_call(
        _oproj_body,
        grid=(HL,),
        in_specs=[
            pl.BlockSpec((1, NR, QR, DH), lambda h: (h, 0, 0, 0)),
            pl.BlockSpec((DH, DM), lambda h: (h, 0)),
        ],
        out_specs=pl.BlockSpec((SQ, DM), lambda h: (0, 0)),
        out_shape=jax.ShapeDtypeStruct((SQ, DM), F32),
    )(o, Wo)

    out = pl.pallas_call(
        _allreduce_body,
        in_specs=[pl.BlockSpec(memory_space=pltpu.VMEM)],
        out_specs=pl.BlockSpec(memory_space=pltpu.VMEM),
        out_shape=jax.ShapeDtypeStruct((1, SQ, DM), F32),
        scratch_shapes=[
            pltpu.VMEM((2, CH, DM), BF16),
            pltpu.SemaphoreType.DMA((2 * (N_DEV - 1),)),
            pltpu.SemaphoreType.DMA((2 * (N_DEV - 1),)),
        ],
        compiler_params=pltpu.CompilerParams(collective_id=1),
    )(part)
    return out


# device time: 375033 ns/iter; 2.8112x vs baseline; 1.0347x over previous
import jax
import jax.numpy as jnp
from jax import lax
from jax.experimental import pallas as pl
from jax.experimental.pallas import tpu as pltpu

N_DEV = 4
SQ = 2048
SKV = 2048
HL = 8
HT = 32
DH = 128
NR = 4
QR = SQ // NR
KR = SKV // NR
DM = 1024
SCALE = 0.08838834764831843
F32 = jnp.float32
BF16 = jnp.bfloat16


def _qproj_body(x_ref, wq_ref, q_ref):
    qh = jnp.dot(x_ref[0], wq_ref[...], preferred_element_type=F32)
    qh = qh.reshape(8, NR, 64, DH).transpose(1, 0, 2, 3)
    q_ref[...] = qh.reshape(NR, 1, QR, DH).astype(BF16)


def _fused_body(q_ref, ks_ref, vs_ref, o_ref,
                l_ref, km_ref, vm_ref, send_sems, recv_sems, loc_sems):
    my = lax.axis_index("i")

    barrier = pltpu.get_barrier_semaphore()
    for p in range(1, N_DEV):
        pl.semaphore_signal(
            barrier, inc=1,
            device_id=((my + p) % N_DEV,),
            device_id_type=pl.DeviceIdType.MESH,
        )
    pl.semaphore_wait(barrier, N_DEV - 1)

    kloc = pltpu.make_async_copy(
        ks_ref.at[:, pl.ds(my * HL, HL)], km_ref.at[:, my], loc_sems.at[0])
    vloc = pltpu.make_async_copy(
        vs_ref.at[:, pl.ds(my * HL, HL)], vm_ref.at[:, my], loc_sems.at[1])
    kloc.start()
    vloc.start()

    def sem_idx(p, t, r):
        return ((p - 1) * 2 + t) * NR + r

    def descriptor(p, t, r, device_id, dst_j):
        s_ref, d_ref = ((ks_ref, km_ref), (vs_ref, vm_ref))[t]
        return pltpu.make_async_remote_copy(
            src_ref=s_ref.at[r, pl.ds(((my + p) % N_DEV) * HL, HL)],
            dst_ref=d_ref.at[r, dst_j],
            send_sem=send_sems.at[sem_idx(p, t, r)],
            recv_sem=recv_sems.at[sem_idx(p, t, r)],
            device_id=(device_id,),
            device_id_type=pl.DeviceIdType.MESH,
        )

    rdmas = []
    for p in range(1, N_DEV):
        for t in range(2):
            for r in range(NR):
                rdma = descriptor(p, t, r, (my + p) % N_DEV, my)
                rdma.start()
                rdmas.append(rdma)

    o_ref[...] = jnp.zeros((NR, HL, QR, DH), F32)
    l_ref[...] = jnp.zeros((NR, HL, QR, 1), F32)

    def consume(j, r):
        def head(h, _):
            q = q_ref[r, h]
            k = km_ref[r, j, h]
            v = vm_ref[r, j, h]
            s = lax.dot_general(
                q, k, (((1,), (1,)), ((), ())), preferred_element_type=F32
            ) * SCALE
            e = jnp.exp(s)
            l_ref[r, h] = l_ref[r, h] + jnp.sum(e, axis=1, keepdims=True)
            o_ref[r, h] = o_ref[r, h] + jnp.dot(
                e.astype(BF16), v, preferred_element_type=F32)
            return 0

        lax.fori_loop(0, HL, head, 0)

    kloc.wait()
    vloc.wait()
    for r in range(NR):
        consume(my, r)
    for p in (1, 3, 2):
        j = (my - p) % N_DEV
        for r in range(NR):
            descriptor(p, 0, r, my, j).wait_recv()
            descriptor(p, 1, r, my, j).wait_recv()
            consume(j, r)

    def normalize(idx, _):
        r = idx // HL
        h = idx % HL
        o_ref[r, h] = o_ref[r, h] / l_ref[r, h]
        return 0

    lax.fori_loop(0, HL * NR, normalize, 0)

    for rdma in rdmas:
        rdma.wait_send()


def _regroup_body(k_ref, v_ref, ks_ref, vs_ref):
    def rg(t):
        t = t.reshape(8, NR, 64, 8, DH).transpose(1, 3, 0, 2, 4)
        return t.reshape(NR, 8, KR, DH).astype(BF16)

    ks_ref[...] = rg(k_ref[0])
    vs_ref[...] = rg(v_ref[0])


def _oproj_body(o_ref, wo_ref, p_ref):
    h = pl.program_id(0)
    ctx = o_ref[:, 0].reshape(NR, 8, 64, DH).transpose(1, 0, 2, 3).reshape(SQ, DH)
    ph = jnp.dot(ctx, wo_ref[...], preferred_element_type=F32)

    @pl.when(h == 0)
    def _():
        p_ref[...] = ph

    @pl.when(h != 0)
    def _():
        p_ref[...] += ph


CH = SQ // N_DEV


def _allreduce_body(p_ref, out_ref, comm_ref, send_sems, recv_sems):
    my = lax.axis_index("i")
    left = (my - 1) % N_DEV
    right = (my + 1) % N_DEV

    barrier = pltpu.get_barrier_semaphore()
    for nbr in (left, right):
        pl.semaphore_signal(
            barrier, inc=1,
            device_id=(nbr,), device_id_type=pl.DeviceIdType.MESH,
        )
    pl.semaphore_wait(barrier, 2)

    def hop(u):
        rdma = pltpu.make_async_remote_copy(
            src_ref=comm_ref.at[u % 2],
            dst_ref=comm_ref.at[(u + 1) % 2],
            send_sem=send_sems.at[u],
            recv_sem=recv_sems.at[u],
            device_id=(right,),
            device_id_type=pl.DeviceIdType.MESH,
        )
        rdma.start()
        rdma.wait()

    def chunk(c):
        return p_ref[pl.ds(c * CH, CH), :]

    def accum(slot, c):
        comm_ref[slot] = (comm_ref[slot].astype(F32) + chunk(c)).astype(BF16)

    comm_ref[0] = chunk((my - 1) % N_DEV).astype(BF16)
    hop(0)
    accum(1, (my - 2) % N_DEV)
    hop(1)
    accum(0, (my - 3) % N_DEV)
    hop(2)
    accum(1, my)
    out_ref[0, pl.ds(my * CH, CH), :] = comm_ref[1].astype(F32)

    for t in range(N_DEV - 1):
        hop(3 + t)
        c = (my - 1 - t) % N_DEV
        out_ref[0, pl.ds(c * CH, CH), :] = comm_ref[t % 2].astype(F32)


def kernel(x, Wq, K_ext, V_ext, Wo):
    ks, vs = pl.pallas_call(
        _regroup_body,
        grid=(N_DEV,),
        in_specs=[
            pl.BlockSpec((1, SKV, HL, DH), lambda g: (0, 0, g, 0)),
            pl.BlockSpec((1, SKV, HL, DH), lambda g: (0, 0, g, 0)),
        ],
        out_specs=[
            pl.BlockSpec((NR, HL, KR, DH), lambda g: (0, g, 0, 0)),
            pl.BlockSpec((NR, HL, KR, DH), lambda g: (0, g, 0, 0)),
        ],
        out_shape=[jax.ShapeDtypeStruct((NR, HT, KR, DH), BF16)] * 2,
        compiler_params=pltpu.CompilerParams(
            vmem_limit_bytes=100 * 1024 * 1024
        ),
    )(K_ext, V_ext)

    q = pl.pallas_call(
        _qproj_body,
        grid=(HL,),
        in_specs=[
            pl.BlockSpec((1, SQ, DM), lambda h: (0, 0, 0)),
            pl.BlockSpec((DM, DH), lambda h: (0, h)),
        ],
        out_specs=pl.BlockSpec((NR, 1, QR, DH), lambda h: (0, h, 0, 0)),
        out_shape=jax.ShapeDtypeStruct((NR, HL, QR, DH), BF16),
        compiler_params=pltpu.CompilerParams(
            vmem_limit_bytes=60 * 1024 * 1024
        ),
    )(x, Wq)

    o = pl.pallas_call(
        _fused_body,
        in_specs=[
            pl.BlockSpec(memory_space=pltpu.VMEM),
            pl.BlockSpec(memory_space=pl.ANY),
            pl.BlockSpec(memory_space=pl.ANY),
        ],
        out_specs=pl.BlockSpec(memory_space=pltpu.VMEM),
        out_shape=jax.ShapeDtypeStruct((NR, HL, QR, DH), F32),
        scratch_shapes=[
            pltpu.VMEM((NR, HL, QR, 1), F32),
            pltpu.VMEM((NR, N_DEV, HL, KR, DH), BF16),
            pltpu.VMEM((NR, N_DEV, HL, KR, DH), BF16),
            pltpu.SemaphoreType.DMA((2 * (N_DEV - 1) * NR,)),
            pltpu.SemaphoreType.DMA((2 * (N_DEV - 1) * NR,)),
            pltpu.SemaphoreType.DMA((2,)),
        ],
        compiler_params=pltpu.CompilerParams(
            collective_id=0, vmem_limit_bytes=60 * 1024 * 1024
        ),
    )(q, ks, vs)

    part = pl.pallas_call(
        _oproj_body,
        grid=(HL,),
        in_specs=[
            pl.BlockSpec((NR, 1, QR, DH), lambda h: (0, h, 0, 0)),
            pl.BlockSpec((DH, DM), lambda h: (h, 0)),
        ],
        out_specs=pl.BlockSpec((SQ, DM), lambda h: (0, 0)),
        out_shape=jax.ShapeDtypeStruct((SQ, DM), F32),
    )(o, Wo)

    out = pl.pallas_call(
        _allreduce_body,
        in_specs=[pl.BlockSpec(memory_space=pltpu.VMEM)],
        out_specs=pl.BlockSpec(memory_space=pltpu.VMEM),
        out_shape=jax.ShapeDtypeStruct((1, SQ, DM), F32),
        scratch_shapes=[
            pltpu.VMEM((2, CH, DM), BF16),
            pltpu.SemaphoreType.DMA((2 * (N_DEV - 1),)),
            pltpu.SemaphoreType.DMA((2 * (N_DEV - 1),)),
        ],
        compiler_params=pltpu.CompilerParams(collective_id=1),
    )(part)
    return out
